# Initial kernel scaffold; baseline (speedup 1.0000x reference)
import jax
import jax.numpy as jnp
from jax.experimental import pallas as pl


def kernel(x_nchw, w1p, b1p, w2p, b2p, wl1p, bl1p, wl2p, bl2p, wl3p, bl3p):
    raise NotImplementedError("write your pallas kernel here")


# speedup vs baseline: 12.3160x; 12.3160x over previous
import jax
import jax.numpy as jnp
from jax.experimental import pallas as pl
from jax.experimental.pallas import tpu as pltpu

_TB = 128
_VMEM_LIMIT = 64 * 1024 * 1024


def _round_up(x, m):
    return -(-x // m) * m


def _net_kernel(x_ref, t1_ref, t2_ref, wb_ref, b2_ref, bl1_ref,
                wl2_ref, bl2_ref, wl3_ref, bl3_ref, o_ref):
    tb = o_ref.shape[0]
    f32 = jnp.float32
    bf16 = jnp.bfloat16

    c1 = jnp.dot(x_ref[...], t1_ref[...], preferred_element_type=f32)
    p = (jnp.maximum(c1[0:tb], 0.0) + jnp.maximum(c1[tb:2 * tb], 0.0)
         + jnp.maximum(c1[2 * tb:3 * tb], 0.0)
         + jnp.maximum(c1[3 * tb:4 * tb], 0.0)) * 0.25

    z = jnp.dot(p.astype(bf16), t2_ref[...], preferred_element_type=f32)
    z = jnp.maximum(z + b2_ref[...], 0.0)

    h = jnp.dot(z.astype(bf16), wb_ref[...], preferred_element_type=f32)
    h = jnp.maximum(h + bl1_ref[...], 0.0)

    h2 = jnp.dot(h.astype(bf16), wl2_ref[...], preferred_element_type=f32)
    h2 = jnp.maximum(h2 + bl2_ref[...], 0.0)

    o_ref[...] = (jnp.dot(h2.astype(bf16), wl3_ref[...],
                          preferred_element_type=f32) + bl3_ref[...])


def _const_index_map(nd):
    return lambda i, _nd=nd: (0,) * _nd


def _prep_weights(w1p, w2p, b2p, wl1p):
    f32 = jnp.float32
    bf16 = jnp.bfloat16
    f = (jnp.arange(14)[None, :, None]
         == jnp.arange(12)[None, None, :] + jnp.arange(3)[:, None, None])
    f = f.astype(f32)
    w1r = w1p.reshape(3, 3, 16)
    t1 = jnp.einsum('aru,bsv,abc->rsuvc', f, f, w1r).reshape(196, 2304)
    e = (jnp.arange(12)[None, :, None]
         == jnp.arange(8)[None, None, :] + 2 * jnp.arange(3)[:, None, None])
    e = e.astype(f32)
    w2r = w2p.reshape(3, 3, 16, 32)
    t2 = jnp.einsum('aui,bvj,abcd->uvcijd', e, e, w2r).reshape(2304, 2048)
    b2t = jnp.tile(b2p.reshape(1, 32), (1, 64))
    ii = jnp.arange(8)
    pos = ((ii[:, None] // 2) * 4 + (ii[None, :] // 2)).reshape(64)
    wbig = (wl1p[pos] * 0.25).reshape(2048, 256)
    return t1.astype(bf16), t2.astype(bf16), b2t, wbig.astype(bf16)


def _prep_x(x_nchw, bp, tb):
    xp = jnp.pad(x_nchw[:, 0], ((0, bp - x_nchw.shape[0]), (1, 1), (1, 1)))
    xp = xp.reshape(bp, 15, 2, 15, 2).transpose(0, 2, 4, 1, 3)
    xm = xp[:, :, :, 0:14, 0:14].reshape(bp, 4, 196)
    nb = bp // tb
    xt = xm.reshape(nb, tb, 4, 196).transpose(0, 2, 1, 3)
    return xt.reshape(nb * 4 * tb, 196).astype(jnp.bfloat16)


def kernel(x_nchw, w1p, b1p, w2p, b2p, wl1p, bl1p, wl2p, bl2p, wl3p, bl3p):
    bsz = x_nchw.shape[0]
    tb = min(_TB, _round_up(bsz, 8))
    bp = _round_up(bsz, tb)
    nb = bp // tb

    t1, t2, b2t, wbig = _prep_weights(w1p, w2p, b2p, wl1p)
    t1 = jnp.concatenate([t1, jnp.tile(b1p.reshape(1, 16), (1, 144))
                          .astype(jnp.bfloat16)], axis=0)
    xt = _prep_x(x_nchw, bp, tb)
    ones = jnp.ones((nb * 4 * tb, 1), jnp.bfloat16)
    xt = jnp.concatenate([xt, ones], axis=1)

    weights = (t1, t2, wbig, b2t, bl1p.astype(jnp.float32),
               wl2p.astype(jnp.bfloat16), bl2p.astype(jnp.float32),
               wl3p.astype(jnp.bfloat16), bl3p.astype(jnp.float32))

    logits = pl.pallas_call(
        _net_kernel,
        out_shape=jax.ShapeDtypeStruct((bp, 128), jnp.float32),
        grid=(nb,),
        in_specs=[pl.BlockSpec((4 * tb, 197), lambda i: (i, 0))]
                 + [pl.BlockSpec(w.shape, _const_index_map(w.ndim))
                    for w in weights],
        out_specs=pl.BlockSpec((tb, 128), lambda i: (i, 0)),
        compiler_params=pltpu.CompilerParams(
            dimension_semantics=("parallel",),
            vmem_limit_bytes=_VMEM_LIMIT),
    )(xt, *weights)
    return logits[:bsz, :10]
